# baseline (speedup 1.0000x reference)
import functools
import jax
import jax.numpy as jnp
from jax import lax
from jax.experimental import pallas as pl
from jax.experimental.pallas import tpu as pltpu
from jax.experimental.pallas import tpu_sc as plsc


_B, _C, _H, _W = 32, 768, 32, 32
_F = 384
_HW = _H * _W
_NW = 32
_RPW = _C // _NW
_CHUNK = _RPW * _HW


def _sc_body(row_hbm, col_hbm, out_hbm, tbl, chunk, sem):
    wid = lax.axis_index("s") * 2 + lax.axis_index("c")
    c0 = wid * _RPW
    is_top = wid < _NW // 2

    @pl.when(is_top)
    def _():
        pltpu.sync_copy(col_hbm, tbl)

    @pl.when(jnp.logical_not(is_top))
    def _():
        pltpu.sync_copy(row_hbm, tbl)

    lane = lax.iota(jnp.int32, 16)
    col0 = jnp.where(is_top, c0, c0 - _F)

    @pl.when(is_top)
    def _():
        for j in range(_RPW):
            cvec = jnp.full((16,), col0 + j, jnp.int32)
            v0 = plsc.load_gather(tbl, [lane * _F + cvec])
            v1 = plsc.load_gather(tbl, [(lane + 16) * _F + cvec])
            for k in range(0, _HW // 16, 2):
                chunk[pl.ds(j * _HW + 16 * k, 16)] = v0
                chunk[pl.ds(j * _HW + 16 * (k + 1), 16)] = v1

    @pl.when(jnp.logical_not(is_top))
    def _():
        for j in range(_RPW):
            cvec = jnp.full((16,), col0 + j, jnp.int32)
            for h in range(_H):
                v = plsc.load_gather(tbl, [cvec + h * _F])
                chunk[pl.ds(j * _HW + 32 * h, 16)] = v
                chunk[pl.ds(j * _HW + 32 * h + 16, 16)] = v

    copies = [
        pltpu.make_async_copy(
            chunk, out_hbm.at[b, pl.ds(c0 * _HW, _CHUNK)], sem)
        for b in range(_B)
    ]
    for c in copies:
        c.start()
    for c in copies:
        c.wait()


def kernel(x, row_embed, col_embed):
    B, C, H, W = x.shape
    mesh = plsc.VectorSubcoreMesh(
        core_axis_name="c", subcore_axis_name="s",
        num_cores=2, num_subcores=16)
    run = functools.partial(
        pl.kernel,
        mesh=mesh,
        compiler_params=pltpu.CompilerParams(
            use_tc_tiling_on_sc=False, needs_layout_passes=False),
        out_type=jax.ShapeDtypeStruct((B, C * _HW), jnp.float32),
        scratch_types=[
            pltpu.VMEM((_H * _F,), jnp.float32),
            pltpu.VMEM((_CHUNK,), jnp.float32),
            pltpu.SemaphoreType.DMA,
        ],
    )(_sc_body)
    out = run(row_embed[:H].reshape(-1), col_embed[:W].reshape(-1))
    return out.reshape(B, C, H, W)


# speedup vs baseline: 13.2288x; 13.2288x over previous
import jax
import jax.numpy as jnp
from jax.experimental import pallas as pl


_B, _C, _H, _W = 32, 768, 32, 32
_F = 384
_HW = _H * _W


def _pos_body(row_ref, col_ref, out_ref):
    col_t = jnp.broadcast_to(
        col_ref[...][None, :, :], (_H, _W, _F)).reshape(_HW, _F)
    row_t = jnp.broadcast_to(
        row_ref[...][:, None, :], (_H, _W, _F)).reshape(_HW, _F)
    out_ref[0, :, :_F] = col_t
    out_ref[0, :, _F:] = row_t


def kernel(x, row_embed, col_embed):
    B, C, H, W = x.shape
    out = pl.pallas_call(
        _pos_body,
        grid=(B,),
        in_specs=[
            pl.BlockSpec((_H, _F), lambda b: (0, 0)),
            pl.BlockSpec((_W, _F), lambda b: (0, 0)),
        ],
        out_specs=pl.BlockSpec((1, _HW, C), lambda b: (b, 0, 0)),
        out_shape=jax.ShapeDtypeStruct((B, _HW, C), jnp.float32),
    )(row_embed[:H], col_embed[:W])
    return out.reshape(B, H, W, C).transpose(0, 3, 1, 2)
